# baseline (speedup 1.0000x reference)
import functools

import jax
import jax.numpy as jnp
from jax import lax
from jax.experimental import pallas as pl
from jax.experimental.pallas import tpu as pltpu
from jax.experimental.pallas import tpu_sc as plsc

N = 10000
E = 320000
D_FEAT = 128
D_EDGE = 16
NT_DIM = 64
ET_DIM = 16

NC = 2
NS = 16
NW = NC * NS

E_PER_W = E // NW
ECHUNK = 1024
N_EFULL = E_PER_W // ECHUNK
ETAIL = E_PER_W - N_EFULL * ECHUNK

NCHUNK = 400
N_NODE_WORKERS = N // NCHUNK


def _gather_rows(table_hbm, idx_ref, dst_ref, rows, sem):
    copies = []
    off = 0
    while off < rows:
        sub = min(128, rows - off)
        copies.append(
            pltpu.async_copy(
                table_hbm.at[idx_ref.at[pl.ds(off, sub)]],
                dst_ref.at[pl.ds(off, sub)],
                sem,
            )
        )
        off += sub
    for c in copies:
        c.wait()


def _edge_chunk(etypes_hbm, etab_hbm, eattr_hbm, ecat_hbm,
                eidx, eemb, eatt, sem, base, rows):
    pltpu.sync_copy(etypes_hbm.at[pl.ds(base, rows)], eidx.at[pl.ds(0, rows)])
    _gather_rows(etab_hbm, eidx, eemb, rows, sem)
    pltpu.sync_copy(eattr_hbm.at[pl.ds(base, rows)], eatt.at[pl.ds(0, rows)])
    pltpu.sync_copy(eemb.at[pl.ds(0, rows)],
                    ecat_hbm.at[pl.ds(base, rows), pl.ds(0, ET_DIM)])
    pltpu.sync_copy(eatt.at[pl.ds(0, rows)],
                    ecat_hbm.at[pl.ds(base, rows), pl.ds(ET_DIM, D_EDGE)])


def _sc_body(x_hbm, eattr_hbm, ntypes_hbm, etypes_hbm, ntab_hbm, etab_hbm,
             xcat_hbm, ecat_hbm,
             nidx, nemb, xbuf, eidx, eemb, eatt, sem):
    c = lax.axis_index("c")
    s = lax.axis_index("s")
    wid = s * NC + c

    @pl.when(wid < N_NODE_WORKERS)
    def _():
        nbase = pl.multiple_of(wid * NCHUNK, 8)
        pltpu.sync_copy(ntypes_hbm.at[pl.ds(nbase, NCHUNK)], nidx)
        _gather_rows(ntab_hbm, nidx, nemb, NCHUNK, sem)
        pltpu.sync_copy(nemb,
                        xcat_hbm.at[pl.ds(nbase, NCHUNK), pl.ds(0, NT_DIM)])
        pltpu.sync_copy(x_hbm.at[pl.ds(nbase, NCHUNK)], xbuf)
        pltpu.sync_copy(xbuf,
                        xcat_hbm.at[pl.ds(nbase, NCHUNK), pl.ds(NT_DIM, D_FEAT)])

    ebase = wid * E_PER_W

    def body(i, carry):
        base = pl.multiple_of(ebase + i * ECHUNK, 8)
        _edge_chunk(etypes_hbm, etab_hbm, eattr_hbm, ecat_hbm,
                    eidx, eemb, eatt, sem, base, ECHUNK)
        return carry

    lax.fori_loop(0, N_EFULL, body, 0)
    _edge_chunk(etypes_hbm, etab_hbm, eattr_hbm, ecat_hbm,
                eidx, eemb, eatt, sem,
                pl.multiple_of(ebase + N_EFULL * ECHUNK, 8), ETAIL)


@jax.jit
def _run(x, eattr, ntypes, etypes, ntab, etab):
    mesh = plsc.VectorSubcoreMesh(core_axis_name="c", subcore_axis_name="s")
    f = pl.kernel(
        _sc_body,
        out_type=[
            jax.ShapeDtypeStruct((N, NT_DIM + D_FEAT), jnp.float32),
            jax.ShapeDtypeStruct((E, ET_DIM + D_EDGE), jnp.float32),
        ],
        mesh=mesh,
        compiler_params=pltpu.CompilerParams(use_tc_tiling_on_sc=False),
        scratch_types=[
            pltpu.VMEM((NCHUNK,), jnp.int32),
            pltpu.VMEM((NCHUNK, NT_DIM), jnp.float32),
            pltpu.VMEM((NCHUNK, D_FEAT), jnp.float32),
            pltpu.VMEM((ECHUNK,), jnp.int32),
            pltpu.VMEM((ECHUNK, ET_DIM), jnp.float32),
            pltpu.VMEM((ECHUNK, D_EDGE), jnp.float32),
            pltpu.SemaphoreType.DMA,
        ],
    )
    return f(x, eattr, ntypes, etypes, ntab, etab)


def kernel(x, eattr, ntypes, etypes, ntype_table, etype_table):
    ntypes = ntypes.astype(jnp.int32)
    etypes = etypes.astype(jnp.int32)
    x_cat, eattr_cat = _run(x, eattr, ntypes, etypes,
                            ntype_table, etype_table)
    return (x_cat, eattr_cat)
